# baseline (speedup 1.0000x reference)
import functools

import jax
import jax.numpy as jnp
from jax.experimental import pallas as pl
from jax.experimental.pallas import tpu as pltpu

EPS = 1e-7


_HI = jax.lax.Precision.HIGHEST


def _eproj_body(ew_ref, we_ref, be_ref, out_ref):
    out_ref[...] = (
        jnp.dot(ew_ref[...], we_ref[...], precision=_HI,
                preferred_element_type=jnp.float32)
        + be_ref[...]
    )


def _eproj(ew, We, be):
    E, ED = ew.shape
    D = We.shape[1]
    BE = 2048
    grid = (pl.cdiv(E, BE),)
    return pl.pallas_call(
        _eproj_body,
        grid=grid,
        in_specs=[
            pl.BlockSpec((BE, ED), lambda i: (i, 0)),
            pl.BlockSpec((ED, D), lambda i: (0, 0)),
            pl.BlockSpec((1, D), lambda i: (0, 0)),
        ],
        out_specs=pl.BlockSpec((BE, D), lambda i: (i, 0)),
        out_shape=jax.ShapeDtypeStruct((E, D), jnp.float32),
    )(ew, We, be.reshape(1, D))


def _mlp_body(x_ref, w1_ref, b1_ref, w2_ref, b2_ref, y_ref):
    h = jnp.dot(x_ref[...], w1_ref[...], precision=_HI,
                preferred_element_type=jnp.float32)
    h = jnp.maximum(h + b1_ref[...], 0.0)
    y_ref[...] = (
        jnp.dot(h, w2_ref[...], precision=_HI,
                preferred_element_type=jnp.float32) + b2_ref[...]
    )


def _mlp(x, W1f, b1f, W2, b2):
    N, D = x.shape
    HID = W1f.shape[1]
    BN = 512
    grid = (pl.cdiv(N, BN),)
    return pl.pallas_call(
        _mlp_body,
        grid=grid,
        in_specs=[
            pl.BlockSpec((BN, D), lambda i: (i, 0)),
            pl.BlockSpec((D, HID), lambda i: (0, 0)),
            pl.BlockSpec((1, HID), lambda i: (0, 0)),
            pl.BlockSpec((HID, D), lambda i: (0, 0)),
            pl.BlockSpec((1, D), lambda i: (0, 0)),
        ],
        out_specs=pl.BlockSpec((BN, D), lambda i: (i, 0)),
        out_shape=jax.ShapeDtypeStruct((N, D), jnp.float32),
    )(x, W1f, b1f.reshape(1, HID), W2, b2.reshape(1, D))


def _edge_phase(x, src, dst, e, t, N):
    msg = jax.nn.relu(x[src] + e) + EPS
    alpha = msg * t
    amax = jax.ops.segment_max(alpha, dst, num_segments=N)
    amax = jnp.where(jnp.isfinite(amax), amax, 0.0)
    ex = jnp.exp(alpha - amax[dst])
    denom = jax.ops.segment_sum(ex, dst, num_segments=N)
    w = ex / (denom[dst] + 1e-16)
    return jax.ops.segment_sum(w * msg, dst, num_segments=N)


def _gen_conv(x, src, dst, e, p):
    N = x.shape[0]
    aggr = _edge_phase(x, src, dst, e, p["t"], N)
    out = aggr + x
    W1f = p["W1"] * p["g1"][None, :]
    b1f = p["b1"] * p["g1"] + p["bt1"]
    return _mlp(out, W1f, b1f, p["W2"], p["b2"])


def kernel(x, edge_index, edge_weight, params):
    src, dst = edge_index[0], edge_index[1]
    e0 = _eproj(edge_weight, params["conv0"]["We"], params["conv0"]["be"])
    h = _gen_conv(x, src, dst, e0, params["conv0"])
    for sp in params["skips"]:
        z = jax.nn.relu(h * sp["gn"] + sp["bn"])
        ek = _eproj(edge_weight, sp["conv"]["We"], sp["conv"]["be"])
        h = h + _gen_conv(z, src, dst, ek, sp["conv"])
    return h


# speedup vs baseline: 2.2312x; 2.2312x over previous
import functools

import jax
import jax.numpy as jnp
from jax import lax
from jax.experimental import pallas as pl
from jax.experimental.pallas import tpu as pltpu
from jax.experimental.pallas import tpu_sc as plsc

EPS = 1e-7
_HI = jax.lax.Precision.HIGHEST

N = 10000
E = 320000
D = 128
HD = 64
ED = 16
HID = 256

NSUB = 16
EP_T = E // NSUB
CH = 128
NCH = EP_T // CH
REM = EP_T - NCH * CH
NPAD = 10240
NP_T = NPAD // NSUB
ZR = 128


def _eproj_body(ew_ref, we_ref, be_ref, *outs):
    ew = ew_ref[...]
    for k in range(3):
        e = jnp.dot(ew, we_ref[k], precision=_HI,
                    preferred_element_type=jnp.float32) + be_ref[0, k]
        outs[2 * k][...] = e[:, :HD]
        outs[2 * k + 1][...] = e[:, HD:]


def _eproj(ew, We3, be3):
    BE = 2048
    outs = [jax.ShapeDtypeStruct((E, HD), jnp.float32) for _ in range(6)]
    return pl.pallas_call(
        _eproj_body,
        grid=(pl.cdiv(E, BE),),
        in_specs=[
            pl.BlockSpec((BE, ED), lambda i: (i, 0)),
            pl.BlockSpec((3, ED, D), lambda i: (0, 0, 0)),
            pl.BlockSpec((1, 3, D), lambda i: (0, 0, 0)),
        ],
        out_specs=[pl.BlockSpec((BE, HD), lambda i: (i, 0))] * 6,
        out_shape=outs,
    )(ew, We3, be3.reshape(1, 3, D))


def _edge_body(src_h, dst_h, e0_h, e1_h, x0_h, x1_h, t_h,
               deno_h, numo_h,
               sidx, didx, ebuf, xbuf, exb, exmb, zb, tb,
               den_s, num_s, sem):
    c = lax.axis_index("c")
    s = lax.axis_index("s")

    def _zrow(i, _):
        for j in range(HD // 16):
            zb[i, pl.ds(16 * j, 16)] = jnp.zeros((16,), jnp.float32)
        return 0
    lax.fori_loop(0, ZR, _zrow, 0)
    for k in range(NP_T // ZR):
        pltpu.sync_copy(zb, den_s.at[pl.ds(s * NP_T + k * ZR, ZR)])
        pltpu.sync_copy(zb, num_s.at[pl.ds(s * NP_T + k * ZR, ZR)])
    pltpu.sync_copy(t_h, tb)
    plsc.subcore_barrier()

    tv = tb[...]

    def _chunk(base, n):
        pltpu.sync_copy(src_h.at[pl.ds(base, n)], sidx.at[pl.ds(0, n)])
        pltpu.sync_copy(dst_h.at[pl.ds(base, n)], didx.at[pl.ds(0, n)])

        @pl.when(c == 0)
        def _():
            pltpu.sync_copy(e0_h.at[pl.ds(base, n)], ebuf.at[pl.ds(0, n)])
            pltpu.async_copy(x0_h.at[sidx.at[pl.ds(0, n)]],
                             xbuf.at[pl.ds(0, n)], sem).wait()

        @pl.when(c == 1)
        def _():
            pltpu.sync_copy(e1_h.at[pl.ds(base, n)], ebuf.at[pl.ds(0, n)])
            pltpu.async_copy(x1_h.at[sidx.at[pl.ds(0, n)]],
                             xbuf.at[pl.ds(0, n)], sem).wait()

        def _row(i, _):
            for j in range(HD // 16):
                sl = pl.ds(16 * j, 16)
                msg = jnp.maximum(xbuf[i, sl] + ebuf[i, sl], 0.0) + EPS
                ex = jnp.exp(msg * tv)
                exb[i, sl] = ex
                exmb[i, sl] = ex * msg
            return 0
        lax.fori_loop(0, n, _row, 0, unroll=2)

        pltpu.sync_copy(exb.at[pl.ds(0, n)],
                        den_s.at[didx.at[pl.ds(0, n)]], add=True)
        pltpu.sync_copy(exmb.at[pl.ds(0, n)],
                        num_s.at[didx.at[pl.ds(0, n)]], add=True)

    def _loop(g, _):
        _chunk(s * EP_T + g * CH, CH)
        return 0
    lax.fori_loop(0, NCH, _loop, 0)
    if REM:
        _chunk(s * EP_T + NCH * CH, REM)

    plsc.subcore_barrier()
    pltpu.sync_copy(den_s.at[pl.ds(s * NP_T, NP_T)],
                    deno_h.at[c, pl.ds(s * NP_T, NP_T)])
    pltpu.sync_copy(num_s.at[pl.ds(s * NP_T, NP_T)],
                    numo_h.at[c, pl.ds(s * NP_T, NP_T)])


def _edge_phase_sc(src, dst, e0, e1, x0, x1, tvec):
    mesh = plsc.VectorSubcoreMesh(core_axis_name="c", subcore_axis_name="s")
    f = pl.kernel(
        _edge_body,
        mesh=mesh,
        compiler_params=pltpu.CompilerParams(use_tc_tiling_on_sc=False),
        out_type=[
            jax.ShapeDtypeStruct((2, NPAD, HD), jnp.float32),
            jax.ShapeDtypeStruct((2, NPAD, HD), jnp.float32),
        ],
        scratch_types=[
            pltpu.VMEM((CH,), jnp.int32),
            pltpu.VMEM((CH,), jnp.int32),
            pltpu.VMEM((CH, HD), jnp.float32),
            pltpu.VMEM((CH, HD), jnp.float32),
            pltpu.VMEM((CH, HD), jnp.float32),
            pltpu.VMEM((CH, HD), jnp.float32),
            pltpu.VMEM((ZR, HD), jnp.float32),
            pltpu.VMEM((16,), jnp.float32),
            pltpu.VMEM_SHARED((NPAD, HD), jnp.float32),
            pltpu.VMEM_SHARED((NPAD, HD), jnp.float32),
            pltpu.SemaphoreType.DMA,
        ],
    )
    return f(src, dst, e0, e1, x0, x1, tvec)


def _fin_body(den_ref, num_ref, xl0_ref, xl1_ref, hprev_ref,
              w1_ref, b1_ref, w2_ref, b2_ref, gb_ref,
              h_ref, z0_ref, z1_ref):
    den = jnp.concatenate([den_ref[0], den_ref[1]], axis=-1)
    num = jnp.concatenate([num_ref[0], num_ref[1]], axis=-1)
    xl = jnp.concatenate([xl0_ref[...], xl1_ref[...]], axis=-1)
    out = num / (den + 1e-16) + xl
    h = jnp.dot(out, w1_ref[...], precision=_HI,
                preferred_element_type=jnp.float32)
    h = jnp.maximum(h + b1_ref[...], 0.0)
    y = jnp.dot(h, w2_ref[...], precision=_HI,
                preferred_element_type=jnp.float32) + b2_ref[...]
    hnew = hprev_ref[...] + y
    h_ref[...] = hnew
    z = jnp.maximum(hnew * gb_ref[0:1] + gb_ref[1:2], 0.0)
    z0_ref[...] = z[:, :HD]
    z1_ref[...] = z[:, HD:]


def _finalize(den, num, xl0, xl1, hprev, W1f, b1f, W2, b2, gn, bn):
    BN = 512
    gb = jnp.stack([gn, bn])
    outs = [
        jax.ShapeDtypeStruct((N, D), jnp.float32),
        jax.ShapeDtypeStruct((N, HD), jnp.float32),
        jax.ShapeDtypeStruct((N, HD), jnp.float32),
    ]
    return pl.pallas_call(
        _fin_body,
        grid=(pl.cdiv(N, BN),),
        in_specs=[
            pl.BlockSpec((2, BN, HD), lambda i: (0, i, 0)),
            pl.BlockSpec((2, BN, HD), lambda i: (0, i, 0)),
            pl.BlockSpec((BN, HD), lambda i: (i, 0)),
            pl.BlockSpec((BN, HD), lambda i: (i, 0)),
            pl.BlockSpec((BN, D), lambda i: (i, 0)),
            pl.BlockSpec((D, HID), lambda i: (0, 0)),
            pl.BlockSpec((1, HID), lambda i: (0, 0)),
            pl.BlockSpec((HID, D), lambda i: (0, 0)),
            pl.BlockSpec((1, D), lambda i: (0, 0)),
            pl.BlockSpec((2, D), lambda i: (0, 0)),
        ],
        out_specs=[
            pl.BlockSpec((BN, D), lambda i: (i, 0)),
            pl.BlockSpec((BN, HD), lambda i: (i, 0)),
            pl.BlockSpec((BN, HD), lambda i: (i, 0)),
        ],
        out_shape=outs,
    )(den, num, xl0, xl1, hprev, W1f, b1f.reshape(1, HID), W2,
      b2.reshape(1, D), gb)


def kernel(x, edge_index, edge_weight, params):
    src = edge_index[0]
    dst = edge_index[1]
    convs = [params["conv0"]] + [sp["conv"] for sp in params["skips"]]
    We3 = jnp.stack([p["We"] for p in convs])
    be3 = jnp.stack([p["be"] for p in convs])
    es = _eproj(edge_weight, We3, be3)

    xl0, xl1 = x[:, :HD], x[:, HD:]
    hprev = jnp.zeros_like(x)
    gns = [params["skips"][0]["gn"], params["skips"][1]["gn"],
           jnp.ones((D,), jnp.float32)]
    bns = [params["skips"][0]["bn"], params["skips"][1]["bn"],
           jnp.zeros((D,), jnp.float32)]

    h = None
    for k, p in enumerate(convs):
        tvec = jnp.broadcast_to(p["t"].astype(jnp.float32), (16,))
        den, num = _edge_phase_sc(src, dst, es[2 * k], es[2 * k + 1],
                                  xl0, xl1, tvec)
        W1f = p["W1"] * p["g1"][None, :]
        b1f = p["b1"] * p["g1"] + p["bt1"]
        h, z0, z1 = _finalize(den, num, xl0, xl1, hprev, W1f, b1f,
                              p["W2"], p["b2"], gns[k], bns[k])
        hprev = h
        xl0, xl1 = z0, z1
    return h


# speedup vs baseline: 2.8007x; 1.2552x over previous
import functools

import jax
import jax.numpy as jnp
from jax import lax
from jax.experimental import pallas as pl
from jax.experimental.pallas import tpu as pltpu
from jax.experimental.pallas import tpu_sc as plsc

EPS = 1e-7
_HI = jax.lax.Precision.HIGHEST

N = 10000
E = 320000
D = 128
HD = 64
ED = 16
HID = 256

NSUB = 16
EP_T = E // NSUB
CH = 80
NCH = EP_T // CH
REM = EP_T - NCH * CH
NPAD = 10240
NP_T = NPAD // NSUB
ZR = 128


def _eproj_body(ew_ref, we_ref, be_ref, *outs):
    ew = ew_ref[...]
    for k in range(3):
        e = jnp.dot(ew, we_ref[k], precision=_HI,
                    preferred_element_type=jnp.float32) + be_ref[0, k]
        outs[2 * k][...] = e[:, :HD]
        outs[2 * k + 1][...] = e[:, HD:]


def _eproj(ew, We3, be3):
    BE = 2048
    outs = [jax.ShapeDtypeStruct((E, HD), jnp.float32) for _ in range(6)]
    return pl.pallas_call(
        _eproj_body,
        grid=(pl.cdiv(E, BE),),
        in_specs=[
            pl.BlockSpec((BE, ED), lambda i: (i, 0)),
            pl.BlockSpec((3, ED, D), lambda i: (0, 0, 0)),
            pl.BlockSpec((1, 3, D), lambda i: (0, 0, 0)),
        ],
        out_specs=[pl.BlockSpec((BE, HD), lambda i: (i, 0))] * 6,
        out_shape=outs,
    )(ew, We3, be3.reshape(1, 3, D))


NB = NCH // 2


def _edge_body(src_h, dst_h, e0_h, e1_h, x0_h, x1_h, t_h, acco_h,
               sidx2, didx2, didxS2, ebuf2, xbuf2, pk2, tb,
               acc_s, semA0, semA1, semX0, semX1, semS0, semS1):
    c = lax.axis_index("c")
    s = lax.axis_index("s")
    tbase = s * EP_T

    semA = (semA0, semA1)
    semX = (semX0, semX1)
    semS = (semS0, semS1)

    def _zrow(i, _):
        for j in range(D // 16):
            pk2[0, i, pl.ds(16 * j, 16)] = jnp.zeros((16,), jnp.float32)
        return 0
    lax.fori_loop(0, CH, _zrow, 0)
    for k in range(NP_T // CH):
        pltpu.sync_copy(pk2.at[0], acc_s.at[pl.ds(s * NP_T + k * CH, CH)])
    pltpu.sync_copy(t_h, tb)
    plsc.subcore_barrier()

    tv = tb[...]

    def issueL(g, p):
        base = tbase + g * CH
        pltpu.async_copy(src_h.at[pl.ds(base, CH)], sidx2.at[p], semA[p])
        pltpu.async_copy(dst_h.at[pl.ds(base, CH)], didx2.at[p], semA[p])

        @pl.when(c == 0)
        def _():
            pltpu.async_copy(e0_h.at[pl.ds(base, CH)], ebuf2.at[p], semA[p])

        @pl.when(c == 1)
        def _():
            pltpu.async_copy(e1_h.at[pl.ds(base, CH)], ebuf2.at[p], semA[p])

    def waitL(p):
        pltpu.make_async_copy(src_h.at[pl.ds(0, CH)], sidx2.at[p], semA[p]).wait()
        pltpu.make_async_copy(dst_h.at[pl.ds(0, CH)], didx2.at[p], semA[p]).wait()
        pltpu.make_async_copy(e0_h.at[pl.ds(0, CH)], ebuf2.at[p], semA[p]).wait()

    def issueG(p):
        @pl.when(c == 0)
        def _():
            pltpu.async_copy(x0_h.at[sidx2.at[p]], xbuf2.at[p], semX[p])

        @pl.when(c == 1)
        def _():
            pltpu.async_copy(x1_h.at[sidx2.at[p]], xbuf2.at[p], semX[p])

    def waitX(p):
        pltpu.make_async_copy(x0_h.at[sidx2.at[p]], xbuf2.at[p], semX[p]).wait()

    def compute(p):
        def _row(i, _):
            for j in range(HD // 16):
                sl = pl.ds(16 * j, 16)
                msg = jnp.maximum(xbuf2[p, i, sl] + ebuf2[p, i, sl], 0.0) + EPS
                ex = jnp.exp(msg * tv)
                pk2[p, i, sl] = ex
                pk2[p, i, pl.ds(HD + 16 * j, 16)] = ex * msg
            return 0
        lax.fori_loop(0, CH, _row, 0, unroll=2)

    def issueS(p):
        for j in range(CH // 16):
            didxS2[p, pl.ds(16 * j, 16)] = didx2[p, pl.ds(16 * j, 16)]
        pltpu.async_copy(pk2.at[p], acc_s.at[didxS2.at[p]], semS[p], add=True)

    def waitS(p):
        pltpu.make_async_copy(pk2.at[p], acc_s.at[didxS2.at[p]], semS[p]).wait()

    issueL(0, 0)
    issueL(1, 1)

    def _iter(i, _):
        waitL(0)
        issueG(0)
        waitL(1)
        issueG(1)

        @pl.when(i > 0)
        def _():
            waitS(0)
        waitX(0)
        compute(0)
        issueS(0)

        @pl.when(i < NB - 1)
        def _():
            issueL(2 * i + 2, 0)

        @pl.when(i > 0)
        def _():
            waitS(1)
        waitX(1)
        compute(1)
        issueS(1)

        @pl.when(i < NB - 1)
        def _():
            issueL(2 * i + 3, 1)
        return 0
    lax.fori_loop(0, NB, _iter, 0)
    waitS(0)
    waitS(1)

    plsc.subcore_barrier()
    pltpu.sync_copy(acc_s.at[pl.ds(s * NP_T, NP_T)],
                    acco_h.at[c, pl.ds(s * NP_T, NP_T)])


def _edge_phase_sc(src, dst, e0, e1, x0, x1, tvec):
    mesh = plsc.VectorSubcoreMesh(core_axis_name="c", subcore_axis_name="s")
    f = pl.kernel(
        _edge_body,
        mesh=mesh,
        compiler_params=pltpu.CompilerParams(use_tc_tiling_on_sc=False),
        out_type=[
            jax.ShapeDtypeStruct((2, NPAD, D), jnp.float32),
        ],
        scratch_types=[
            pltpu.VMEM((2, CH), jnp.int32),
            pltpu.VMEM((2, CH), jnp.int32),
            pltpu.VMEM((2, CH), jnp.int32),
            pltpu.VMEM((2, CH, HD), jnp.float32),
            pltpu.VMEM((2, CH, HD), jnp.float32),
            pltpu.VMEM((2, CH, D), jnp.float32),
            pltpu.VMEM((16,), jnp.float32),
            pltpu.VMEM_SHARED((NPAD, D), jnp.float32),
            pltpu.SemaphoreType.DMA,
            pltpu.SemaphoreType.DMA,
            pltpu.SemaphoreType.DMA,
            pltpu.SemaphoreType.DMA,
            pltpu.SemaphoreType.DMA,
            pltpu.SemaphoreType.DMA,
        ],
    )
    (acc,) = f(src, dst, e0, e1, x0, x1, tvec)
    return acc


def _fin_body(acc_ref, xl0_ref, xl1_ref, hprev_ref,
              w1_ref, b1_ref, w2_ref, b2_ref, gb_ref,
              h_ref, z0_ref, z1_ref):
    den = jnp.concatenate([acc_ref[0, :, :HD], acc_ref[1, :, :HD]], axis=-1)
    num = jnp.concatenate([acc_ref[0, :, HD:], acc_ref[1, :, HD:]], axis=-1)
    xl = jnp.concatenate([xl0_ref[...], xl1_ref[...]], axis=-1)
    out = num / (den + 1e-16) + xl
    h = jnp.dot(out, w1_ref[...], precision=_HI,
                preferred_element_type=jnp.float32)
    h = jnp.maximum(h + b1_ref[...], 0.0)
    y = jnp.dot(h, w2_ref[...], precision=_HI,
                preferred_element_type=jnp.float32) + b2_ref[...]
    hnew = hprev_ref[...] + y
    h_ref[...] = hnew
    z = jnp.maximum(hnew * gb_ref[0:1] + gb_ref[1:2], 0.0)
    z0_ref[...] = z[:, :HD]
    z1_ref[...] = z[:, HD:]


def _finalize(acc, xl0, xl1, hprev, W1f, b1f, W2, b2, gn, bn):
    BN = 512
    gb = jnp.stack([gn, bn])
    outs = [
        jax.ShapeDtypeStruct((N, D), jnp.float32),
        jax.ShapeDtypeStruct((N, HD), jnp.float32),
        jax.ShapeDtypeStruct((N, HD), jnp.float32),
    ]
    return pl.pallas_call(
        _fin_body,
        grid=(pl.cdiv(N, BN),),
        in_specs=[
            pl.BlockSpec((2, BN, D), lambda i: (0, i, 0)),
            pl.BlockSpec((BN, HD), lambda i: (i, 0)),
            pl.BlockSpec((BN, HD), lambda i: (i, 0)),
            pl.BlockSpec((BN, D), lambda i: (i, 0)),
            pl.BlockSpec((D, HID), lambda i: (0, 0)),
            pl.BlockSpec((1, HID), lambda i: (0, 0)),
            pl.BlockSpec((HID, D), lambda i: (0, 0)),
            pl.BlockSpec((1, D), lambda i: (0, 0)),
            pl.BlockSpec((2, D), lambda i: (0, 0)),
        ],
        out_specs=[
            pl.BlockSpec((BN, D), lambda i: (i, 0)),
            pl.BlockSpec((BN, HD), lambda i: (i, 0)),
            pl.BlockSpec((BN, HD), lambda i: (i, 0)),
        ],
        out_shape=outs,
    )(acc, xl0, xl1, hprev, W1f, b1f.reshape(1, HID), W2,
      b2.reshape(1, D), gb)


def kernel(x, edge_index, edge_weight, params):
    src = edge_index[0]
    dst = edge_index[1]
    convs = [params["conv0"]] + [sp["conv"] for sp in params["skips"]]
    We3 = jnp.stack([p["We"] for p in convs])
    be3 = jnp.stack([p["be"] for p in convs])
    es = _eproj(edge_weight, We3, be3)

    xl0, xl1 = x[:, :HD], x[:, HD:]
    hprev = jnp.zeros_like(x)
    gns = [params["skips"][0]["gn"], params["skips"][1]["gn"],
           jnp.ones((D,), jnp.float32)]
    bns = [params["skips"][0]["bn"], params["skips"][1]["bn"],
           jnp.zeros((D,), jnp.float32)]

    h = None
    for k, p in enumerate(convs):
        tvec = jnp.broadcast_to(p["t"].astype(jnp.float32), (16,))
        acc = _edge_phase_sc(src, dst, es[2 * k], es[2 * k + 1],
                             xl0, xl1, tvec)
        W1f = p["W1"] * p["g1"][None, :]
        b1f = p["b1"] * p["g1"] + p["bt1"]
        h, z0, z1 = _finalize(acc, xl0, xl1, hprev, W1f, b1f,
                              p["W2"], p["b2"], gns[k], bns[k])
        hprev = h
        xl0, xl1 = z0, z1
    return h


# speedup vs baseline: 2.8211x; 1.0073x over previous
import functools

import jax
import jax.numpy as jnp
from jax import lax
from jax.experimental import pallas as pl
from jax.experimental.pallas import tpu as pltpu
from jax.experimental.pallas import tpu_sc as plsc

EPS = 1e-7
_HI = jax.lax.Precision.HIGHEST

N = 10000
E = 320000
D = 128
HD = 64
ED = 16
HID = 256

NSUB = 16
EP_T = E // NSUB
CH = 80
NCH = EP_T // CH
REM = EP_T - NCH * CH
NPAD = 10240
NP_T = NPAD // NSUB
ZR = 128


def _eproj_body(ew_ref, we_ref, be_ref, *outs):
    ew = ew_ref[...]
    for k in range(3):
        e = jnp.dot(ew, we_ref[k], precision=_HI,
                    preferred_element_type=jnp.float32) + be_ref[0, k]
        outs[2 * k][...] = e[:, :HD]
        outs[2 * k + 1][...] = e[:, HD:]


def _eproj(ew, We3, be3):
    BE = 2048
    outs = [jax.ShapeDtypeStruct((E, HD), jnp.float32) for _ in range(6)]
    return pl.pallas_call(
        _eproj_body,
        grid=(pl.cdiv(E, BE),),
        in_specs=[
            pl.BlockSpec((BE, ED), lambda i: (i, 0)),
            pl.BlockSpec((3, ED, D), lambda i: (0, 0, 0)),
            pl.BlockSpec((1, 3, D), lambda i: (0, 0, 0)),
        ],
        out_specs=[pl.BlockSpec((BE, HD), lambda i: (i, 0))] * 6,
        out_shape=outs,
    )(ew, We3, be3.reshape(1, 3, D))


NB = NCH // 2


def _edge_body(ei_h, e0_h, e1_h, x0_h, x1_h, t_h, acco_h,
               sd2, didxS2, ebuf2, xbuf2, pk2, tb,
               acc_s, semA0, semA1, semX0, semX1, semS0, semS1):
    c = lax.axis_index("c")
    s = lax.axis_index("s")
    tbase = s * EP_T

    semA = (semA0, semA1)
    semX = (semX0, semX1)
    semS = (semS0, semS1)

    def _zrow(i, _):
        for j in range(D // 16):
            pk2[0, i, pl.ds(16 * j, 16)] = jnp.zeros((16,), jnp.float32)
        return 0
    lax.fori_loop(0, CH, _zrow, 0)
    for k in range(NP_T // CH):
        pltpu.sync_copy(pk2.at[0], acc_s.at[pl.ds(s * NP_T + k * CH, CH)])
    pltpu.sync_copy(t_h, tb)
    plsc.subcore_barrier()

    tv = tb[...]

    def issueL(g, p):
        base = tbase + g * CH
        pltpu.async_copy(ei_h.at[:, pl.ds(base, CH)], sd2.at[p], semA[p])

        @pl.when(c == 0)
        def _():
            pltpu.async_copy(e0_h.at[pl.ds(base, CH)], ebuf2.at[p], semA[p])

        @pl.when(c == 1)
        def _():
            pltpu.async_copy(e1_h.at[pl.ds(base, CH)], ebuf2.at[p], semA[p])

    def waitL(p):
        pltpu.make_async_copy(ei_h.at[:, pl.ds(0, CH)], sd2.at[p], semA[p]).wait()
        pltpu.make_async_copy(e0_h.at[pl.ds(0, CH)], ebuf2.at[p], semA[p]).wait()

    def issueG(p):
        @pl.when(c == 0)
        def _():
            pltpu.async_copy(x0_h.at[sd2.at[p, 0]], xbuf2.at[p], semX[p])

        @pl.when(c == 1)
        def _():
            pltpu.async_copy(x1_h.at[sd2.at[p, 0]], xbuf2.at[p], semX[p])

    def waitX(p):
        pltpu.make_async_copy(x0_h.at[sd2.at[p, 0]], xbuf2.at[p], semX[p]).wait()

    def compute(p):
        def _row(i, _):
            for j in range(HD // 16):
                sl = pl.ds(16 * j, 16)
                msg = jnp.maximum(xbuf2[p, i, sl] + ebuf2[p, i, sl], 0.0) + EPS
                ex = jnp.exp(msg * tv)
                pk2[p, i, sl] = ex
                pk2[p, i, pl.ds(HD + 16 * j, 16)] = ex * msg
            return 0
        lax.fori_loop(0, CH, _row, 0, unroll=4)

    def issueS(p):
        for j in range(CH // 16):
            didxS2[p, pl.ds(16 * j, 16)] = sd2[p, 1, pl.ds(16 * j, 16)]
        pltpu.async_copy(pk2.at[p], acc_s.at[didxS2.at[p]], semS[p], add=True)

    def waitS(p):
        pltpu.make_async_copy(pk2.at[p], acc_s.at[didxS2.at[p]], semS[p]).wait()

    issueL(0, 0)
    issueL(1, 1)

    def _iter(i, _):
        waitL(0)
        issueG(0)
        waitL(1)
        issueG(1)

        @pl.when(i > 0)
        def _():
            waitS(0)
        waitX(0)
        compute(0)
        issueS(0)

        @pl.when(i < NB - 1)
        def _():
            issueL(2 * i + 2, 0)

        @pl.when(i > 0)
        def _():
            waitS(1)
        waitX(1)
        compute(1)
        issueS(1)

        @pl.when(i < NB - 1)
        def _():
            issueL(2 * i + 3, 1)
        return 0
    lax.fori_loop(0, NB, _iter, 0)
    waitS(0)
    waitS(1)

    plsc.subcore_barrier()
    pltpu.sync_copy(acc_s.at[pl.ds(s * NP_T, NP_T)],
                    acco_h.at[c, pl.ds(s * NP_T, NP_T)])


def _edge_phase_sc(ei, e0, e1, x0, x1, tvec):
    mesh = plsc.VectorSubcoreMesh(core_axis_name="c", subcore_axis_name="s")
    f = pl.kernel(
        _edge_body,
        mesh=mesh,
        compiler_params=pltpu.CompilerParams(use_tc_tiling_on_sc=False),
        out_type=[
            jax.ShapeDtypeStruct((2, NPAD, D), jnp.float32),
        ],
        scratch_types=[
            pltpu.VMEM((2, 2, CH), jnp.int32),
            pltpu.VMEM((2, CH), jnp.int32),
            pltpu.VMEM((2, CH, HD), jnp.float32),
            pltpu.VMEM((2, CH, HD), jnp.float32),
            pltpu.VMEM((2, CH, D), jnp.float32),
            pltpu.VMEM((16,), jnp.float32),
            pltpu.VMEM_SHARED((NPAD, D), jnp.float32),
            pltpu.SemaphoreType.DMA,
            pltpu.SemaphoreType.DMA,
            pltpu.SemaphoreType.DMA,
            pltpu.SemaphoreType.DMA,
            pltpu.SemaphoreType.DMA,
            pltpu.SemaphoreType.DMA,
        ],
    )
    (acc,) = f(ei, e0, e1, x0, x1, tvec)
    return acc


def _fin_body(acc_ref, xl0_ref, xl1_ref, hprev_ref,
              w1_ref, b1_ref, w2_ref, b2_ref, gb_ref,
              h_ref, z0_ref, z1_ref):
    den = jnp.concatenate([acc_ref[0, :, :HD], acc_ref[1, :, :HD]], axis=-1)
    num = jnp.concatenate([acc_ref[0, :, HD:], acc_ref[1, :, HD:]], axis=-1)
    xl = jnp.concatenate([xl0_ref[...], xl1_ref[...]], axis=-1)
    out = num / (den + 1e-16) + xl
    h = jnp.dot(out, w1_ref[...], precision=_HI,
                preferred_element_type=jnp.float32)
    h = jnp.maximum(h + b1_ref[...], 0.0)
    y = jnp.dot(h, w2_ref[...], precision=_HI,
                preferred_element_type=jnp.float32) + b2_ref[...]
    hnew = hprev_ref[...] + y
    h_ref[...] = hnew
    z = jnp.maximum(hnew * gb_ref[0:1] + gb_ref[1:2], 0.0)
    z0_ref[...] = z[:, :HD]
    z1_ref[...] = z[:, HD:]


def _finalize(acc, xl0, xl1, hprev, W1f, b1f, W2, b2, gn, bn):
    BN = 512
    gb = jnp.stack([gn, bn])
    outs = [
        jax.ShapeDtypeStruct((N, D), jnp.float32),
        jax.ShapeDtypeStruct((N, HD), jnp.float32),
        jax.ShapeDtypeStruct((N, HD), jnp.float32),
    ]
    return pl.pallas_call(
        _fin_body,
        grid=(pl.cdiv(N, BN),),
        in_specs=[
            pl.BlockSpec((2, BN, D), lambda i: (0, i, 0)),
            pl.BlockSpec((BN, HD), lambda i: (i, 0)),
            pl.BlockSpec((BN, HD), lambda i: (i, 0)),
            pl.BlockSpec((BN, D), lambda i: (i, 0)),
            pl.BlockSpec((D, HID), lambda i: (0, 0)),
            pl.BlockSpec((1, HID), lambda i: (0, 0)),
            pl.BlockSpec((HID, D), lambda i: (0, 0)),
            pl.BlockSpec((1, D), lambda i: (0, 0)),
            pl.BlockSpec((2, D), lambda i: (0, 0)),
        ],
        out_specs=[
            pl.BlockSpec((BN, D), lambda i: (i, 0)),
            pl.BlockSpec((BN, HD), lambda i: (i, 0)),
            pl.BlockSpec((BN, HD), lambda i: (i, 0)),
        ],
        out_shape=outs,
    )(acc, xl0, xl1, hprev, W1f, b1f.reshape(1, HID), W2,
      b2.reshape(1, D), gb)


def kernel(x, edge_index, edge_weight, params):
    src = edge_index[0]
    dst = edge_index[1]
    convs = [params["conv0"]] + [sp["conv"] for sp in params["skips"]]
    We3 = jnp.stack([p["We"] for p in convs])
    be3 = jnp.stack([p["be"] for p in convs])
    es = _eproj(edge_weight, We3, be3)

    xl0, xl1 = x[:, :HD], x[:, HD:]
    hprev = jnp.zeros_like(x)
    gns = [params["skips"][0]["gn"], params["skips"][1]["gn"],
           jnp.ones((D,), jnp.float32)]
    bns = [params["skips"][0]["bn"], params["skips"][1]["bn"],
           jnp.zeros((D,), jnp.float32)]

    h = None
    for k, p in enumerate(convs):
        tvec = jnp.broadcast_to(p["t"].astype(jnp.float32), (16,))
        acc = _edge_phase_sc(edge_index, es[2 * k], es[2 * k + 1],
                             xl0, xl1, tvec)
        W1f = p["W1"] * p["g1"][None, :]
        b1f = p["b1"] * p["g1"] + p["bt1"]
        h, z0, z1 = _finalize(acc, xl0, xl1, hprev, W1f, b1f,
                              p["W2"], p["b2"], gns[k], bns[k])
        hprev = h
        xl0, xl1 = z0, z1
    return h
